# Initial kernel scaffold; baseline (speedup 1.0000x reference)
import jax
import jax.numpy as jnp
from jax.experimental import pallas as pl


def kernel(indices, tables, bias):
    raise NotImplementedError("write your pallas kernel here")


# speedup vs baseline: 1.0049x; 1.0049x over previous
import jax
import jax.numpy as jnp
from jax import lax
from jax.experimental import pallas as pl
from jax.experimental.pallas import tpu as pltpu
from jax.experimental.pallas import tpu_sc as plsc

B = 16384
F = 26
V = 1000000

NC = 2
NS = 16
NW = NC * NS
BPW = B // NW
LANES = 16
ROWS_PER_FIELD = V // LANES
CHUNK = 128
NCHUNK = BPW // CHUNK
NVEC = BPW // LANES


def _body(idx_hbm, tab_hbm, bias_hbm, out_hbm, idx_v, rowid_v, rows_v, out_v,
          bias_v, sem):
    wid = lax.axis_index("s") * NC + lax.axis_index("c")
    base = wid * BPW

    pltpu.sync_copy(idx_hbm.at[:, pl.ds(base, BPW)], idx_v)
    pltpu.sync_copy(bias_hbm, bias_v)

    b_vec = bias_v[...]

    def init(c, carry):
        out_v[pl.ds(c * LANES, LANES)] = b_vec
        return carry

    lax.fori_loop(0, NVEC, init, 0)

    lane_iota = lax.iota(jnp.int32, LANES)

    def field_body(f, carry):
        row_base = f * ROWS_PER_FIELD

        def rid(c, carry2):
            iv = idx_v[f, pl.ds(c * LANES, LANES)]
            rowid_v[pl.ds(c * LANES, LANES)] = (iv >> 4) + row_base
            return carry2

        lax.fori_loop(0, NVEC, rid, 0)

        for q in range(NCHUNK):
            pltpu.make_async_copy(
                tab_hbm.at[rowid_v.at[pl.ds(q * CHUNK, CHUNK)]],
                rows_v.at[pl.ds(q * CHUNK, CHUNK), :],
                sem,
            ).start()
        pltpu.make_async_copy(tab_hbm.at[pl.ds(0, BPW), :], rows_v, sem).wait()

        def sel(c, carry2):
            iv = idx_v[f, pl.ds(c * LANES, LANES)]
            pos = lane_iota + c * LANES
            v = plsc.load_gather(rows_v, [pos, iv & 15])
            acc = out_v[pl.ds(c * LANES, LANES)]
            out_v[pl.ds(c * LANES, LANES)] = acc + v
            return carry2

        lax.fori_loop(0, NVEC, sel, 0)
        return carry

    lax.fori_loop(0, F, field_body, 0)

    pltpu.sync_copy(out_v, out_hbm.at[pl.ds(base, BPW)])


@jax.jit
def _lookup_sum(idx_t, tab_rows, bias16):
    mesh = plsc.VectorSubcoreMesh(core_axis_name="c", subcore_axis_name="s")
    return pl.kernel(
        _body,
        out_type=jax.ShapeDtypeStruct((B,), jnp.float32),
        mesh=mesh,
        scratch_types=[
            pltpu.VMEM((F, BPW), jnp.int32),
            pltpu.VMEM((BPW,), jnp.int32),
            pltpu.VMEM((BPW, LANES), jnp.float32),
            pltpu.VMEM((BPW,), jnp.float32),
            pltpu.VMEM((LANES,), jnp.float32),
            pltpu.SemaphoreType.DMA,
        ],
        compiler_params=pltpu.CompilerParams(
            needs_layout_passes=False, use_tc_tiling_on_sc=False),
    )(idx_t, tab_rows, bias16)


def kernel(indices, tables, bias):
    idx_t = indices.astype(jnp.int32).T
    tab_rows = tables.reshape(F * ROWS_PER_FIELD, LANES)
    bias16 = jnp.broadcast_to(bias.astype(jnp.float32), (LANES,))
    return _lookup_sum(idx_t, tab_rows, bias16)


# speedup vs baseline: 8.2571x; 8.2169x over previous
import jax
import jax.numpy as jnp
from jax import lax
from jax.experimental import pallas as pl
from jax.experimental.pallas import tpu as pltpu
from jax.experimental.pallas import tpu_sc as plsc

B = 16384
F = 26
V = 1000000

NC = 2
NS = 16
NW = NC * NS
BPW = B // NW
LANES = 16
ROWLEN = 128
FPAD = 32
VPAD = 1000064
CTILES = VPAD // ROWLEN
NROWS = FPAD * CTILES
CHUNK = 128
NCHUNK = BPW // CHUNK
NGATHER = F * NCHUNK
VPC = CHUNK // LANES


def _body(idx_hbm, tab_hbm, bias_hbm, out_hbm, idx_v, rowid_v, rows_v, out_v,
          bias_v, sem):
    wid = lax.axis_index("s") * NC + lax.axis_index("c")
    base = wid * BPW

    pltpu.sync_copy(idx_hbm.at[:, pl.ds(base, BPW)], idx_v)
    pltpu.sync_copy(bias_hbm, bias_v)

    b_vec = bias_v[...]

    def init(c, carry):
        out_v[pl.ds(c * LANES, LANES)] = b_vec
        return carry

    lax.fori_loop(0, BPW // LANES, init, 0)

    lane_iota = lax.iota(jnp.int32, LANES)

    def chunk_body(j, carry):
        f = j // NCHUNK
        q = j % NCHUNK
        foff = (f >> 3) * (8 * CTILES) + (f & 7)

        def rid(k, carry2):
            iv = idx_v[f, pl.ds(q * CHUNK + k * LANES, LANES)]
            rowid_v[pl.ds(k * LANES, LANES)] = ((iv >> 7) << 3) + foff
            return carry2

        lax.fori_loop(0, VPC, rid, 0)

        pltpu.make_async_copy(
            tab_hbm.at[rowid_v.at[pl.ds(0, CHUNK)]],
            rows_v,
            sem,
        ).start()
        pltpu.make_async_copy(tab_hbm.at[pl.ds(0, CHUNK), :], rows_v, sem).wait()

        def sel(k, carry2):
            col = q * CHUNK + k * LANES
            iv = idx_v[f, pl.ds(col, LANES)]
            pos = lane_iota + k * LANES
            v = plsc.load_gather(rows_v, [pos, iv & 127])
            acc = out_v[pl.ds(col, LANES)]
            out_v[pl.ds(col, LANES)] = acc + v
            return carry2

        lax.fori_loop(0, VPC, sel, 0)
        return carry

    lax.fori_loop(0, NGATHER, chunk_body, 0)

    pltpu.sync_copy(out_v, out_hbm.at[pl.ds(base, BPW)])


@jax.jit
def _lookup_sum(idx_t, tab_rows, bias16):
    mesh = plsc.VectorSubcoreMesh(core_axis_name="c", subcore_axis_name="s")
    return pl.kernel(
        _body,
        out_type=jax.ShapeDtypeStruct((B,), jnp.float32),
        mesh=mesh,
        scratch_types=[
            pltpu.VMEM((F, BPW), jnp.int32),
            pltpu.VMEM((CHUNK,), jnp.int32),
            pltpu.VMEM((CHUNK, ROWLEN), jnp.float32),
            pltpu.VMEM((BPW,), jnp.float32),
            pltpu.VMEM((LANES,), jnp.float32),
            pltpu.SemaphoreType.DMA,
        ],
        compiler_params=pltpu.CompilerParams(
            needs_layout_passes=False, use_tc_tiling_on_sc=True),
    )(idx_t, tab_rows, bias16)


def kernel(indices, tables, bias):
    idx_t = indices.astype(jnp.int32).T
    tab_pad = jnp.pad(tables, ((0, FPAD - F), (0, VPAD - V)))
    tab_rows = (
        tab_pad.reshape(FPAD // 8, 8, CTILES, ROWLEN)
        .transpose(0, 2, 1, 3)
        .reshape(NROWS, ROWLEN)
    )
    bias16 = jnp.broadcast_to(bias.astype(jnp.float32), (LANES,))
    return _lookup_sum(idx_t, tab_rows, bias16)


# speedup vs baseline: 10.9219x; 1.3227x over previous
import jax
import jax.numpy as jnp
from jax import lax
from jax.experimental import pallas as pl
from jax.experimental.pallas import tpu as pltpu
from jax.experimental.pallas import tpu_sc as plsc

B = 16384
F = 26
V = 1000000

NC = 2
NS = 16
NW = NC * NS
BPW = B // NW
LANES = 16
ROWLEN = 128
FPAD = 32
VPAD = 1000064
CTILES = VPAD // ROWLEN
NROWS = FPAD * CTILES
NROWS64 = NROWS * 8
CHUNK = 128
NCHUNK = BPW // CHUNK
NGATHER = F * NCHUNK
VPC = CHUNK // LANES


def _body(idx_hbm, tab_hbm, bias_hbm, out_hbm, idx_v, rowid_v, rows_v, out_v,
          bias_v, sem):
    wid = lax.axis_index("s") * NC + lax.axis_index("c")
    base = wid * BPW

    pltpu.sync_copy(idx_hbm.at[:, pl.ds(base, BPW)], idx_v)
    pltpu.sync_copy(bias_hbm, bias_v)

    b_vec = bias_v[...]

    def init(c, carry):
        out_v[pl.ds(c * LANES, LANES)] = b_vec
        return carry

    lax.fori_loop(0, BPW // LANES, init, 0)

    lane_iota = lax.iota(jnp.int32, LANES)

    def chunk_body(j, carry):
        f = j // NCHUNK
        q = j % NCHUNK
        foff = ((f >> 3) * (8 * CTILES) + (f & 7)) * 8

        def rid(k, carry2):
            iv = idx_v[f, pl.ds(q * CHUNK + k * LANES, LANES)]
            rowid_v[pl.ds(k * LANES, LANES)] = (
                ((iv >> 7) << 6) + ((iv >> 4) & 7) + foff)
            return carry2

        lax.fori_loop(0, VPC, rid, 0)

        pltpu.make_async_copy(
            tab_hbm.at[rowid_v.at[pl.ds(0, CHUNK)]],
            rows_v,
            sem,
        ).start()
        pltpu.make_async_copy(tab_hbm.at[pl.ds(0, CHUNK), :], rows_v, sem).wait()

        def sel(k, carry2):
            col = q * CHUNK + k * LANES
            iv = idx_v[f, pl.ds(col, LANES)]
            pos = lane_iota + k * LANES
            v = plsc.load_gather(rows_v, [pos, iv & 15])
            acc = out_v[pl.ds(col, LANES)]
            out_v[pl.ds(col, LANES)] = acc + v
            return carry2

        lax.fori_loop(0, VPC, sel, 0)
        return carry

    lax.fori_loop(0, NGATHER, chunk_body, 0)

    pltpu.sync_copy(out_v, out_hbm.at[pl.ds(base, BPW)])


@jax.jit
def _lookup_sum(idx_t, tab_rows, bias16):
    mesh = plsc.VectorSubcoreMesh(core_axis_name="c", subcore_axis_name="s")
    return pl.kernel(
        _body,
        out_type=jax.ShapeDtypeStruct((B,), jnp.float32),
        mesh=mesh,
        scratch_types=[
            pltpu.VMEM((F, BPW), jnp.int32),
            pltpu.VMEM((CHUNK,), jnp.int32),
            pltpu.VMEM((CHUNK, LANES), jnp.float32),
            pltpu.VMEM((BPW,), jnp.float32),
            pltpu.VMEM((LANES,), jnp.float32),
            pltpu.SemaphoreType.DMA,
        ],
        compiler_params=pltpu.CompilerParams(
            needs_layout_passes=False, use_tc_tiling_on_sc=False),
    )(idx_t, tab_rows, bias16)


def kernel(indices, tables, bias):
    idx_t = indices.astype(jnp.int32).T
    tab_pad = jnp.pad(tables, ((0, FPAD - F), (0, VPAD - V)))
    tab_rows = (
        tab_pad.reshape(FPAD // 8, 8, CTILES, ROWLEN)
        .transpose(0, 2, 1, 3)
        .reshape(NROWS64, LANES)
    )
    bias16 = jnp.broadcast_to(bias.astype(jnp.float32), (LANES,))
    return _lookup_sum(idx_t, tab_rows, bias16)


# speedup vs baseline: 16.7464x; 1.5333x over previous
import jax
import jax.numpy as jnp
from jax import lax
from jax.experimental import pallas as pl
from jax.experimental.pallas import tpu as pltpu
from jax.experimental.pallas import tpu_sc as plsc

B = 16384
F = 26
V = 1000000

NC = 2
NS = 16
NW = NC * NS
BPW = B // NW
LANES = 16
ROWLEN = 128
FPAD = 32
VPAD = 1000064
CTILES = VPAD // ROWLEN
NROWS = FPAD * CTILES
NROWS64 = NROWS * 8
CHUNK = 128
NCHUNK = BPW // CHUNK
NGATHER = F * NCHUNK
VPC = CHUNK // LANES
DEPTH = 4


def _body(idx_hbm, tab_hbm, bias_hbm, out_hbm, idx_v, rowid_v, rows_v, out_v,
          bias_v, sem):
    wid = lax.axis_index("s") * NC + lax.axis_index("c")
    base = wid * BPW

    pltpu.sync_copy(idx_hbm.at[:, pl.ds(base, BPW)], idx_v)
    pltpu.sync_copy(bias_hbm, bias_v)

    b_vec = bias_v[...]

    def init(c, carry):
        out_v[pl.ds(c * LANES, LANES)] = b_vec
        return carry

    lax.fori_loop(0, BPW // LANES, init, 0)

    lane_iota = lax.iota(jnp.int32, LANES)

    def fire(j):
        f = j // NCHUNK
        q = j % NCHUNK
        slot = j % DEPTH
        foff = ((f >> 3) * (8 * CTILES) + (f & 7)) * 8

        def rid(k, carry2):
            iv = idx_v[f, pl.ds(q * CHUNK + k * LANES, LANES)]
            rowid_v[slot, pl.ds(k * LANES, LANES)] = (
                ((iv >> 7) << 6) + ((iv >> 4) & 7) + foff)
            return carry2

        lax.fori_loop(0, VPC, rid, 0)
        pltpu.make_async_copy(
            tab_hbm.at[rowid_v.at[slot]],
            rows_v.at[slot],
            sem.at[slot],
        ).start()

    for d in range(DEPTH):
        fire(d)

    def chunk_body(j, carry):
        slot = j % DEPTH
        f = j // NCHUNK
        q = j % NCHUNK
        pltpu.make_async_copy(
            tab_hbm.at[pl.ds(0, CHUNK), :], rows_v.at[slot], sem.at[slot]
        ).wait()

        def sel(k, carry2):
            col = q * CHUNK + k * LANES
            iv = idx_v[f, pl.ds(col, LANES)]
            pos = lane_iota + k * LANES
            v = plsc.load_gather(rows_v.at[slot], [pos, iv & 15])
            acc = out_v[pl.ds(col, LANES)]
            out_v[pl.ds(col, LANES)] = acc + v
            return carry2

        lax.fori_loop(0, VPC, sel, 0)

        @pl.when(j + DEPTH < NGATHER)
        def _():
            fire(j + DEPTH)

        return carry

    lax.fori_loop(0, NGATHER, chunk_body, 0)

    pltpu.sync_copy(out_v, out_hbm.at[pl.ds(base, BPW)])


@jax.jit
def _lookup_sum(idx_t, tab_rows, bias16):
    mesh = plsc.VectorSubcoreMesh(core_axis_name="c", subcore_axis_name="s")
    return pl.kernel(
        _body,
        out_type=jax.ShapeDtypeStruct((B,), jnp.float32),
        mesh=mesh,
        scratch_types=[
            pltpu.VMEM((F, BPW), jnp.int32),
            pltpu.VMEM((DEPTH, CHUNK), jnp.int32),
            pltpu.VMEM((DEPTH, CHUNK, LANES), jnp.float32),
            pltpu.VMEM((BPW,), jnp.float32),
            pltpu.VMEM((LANES,), jnp.float32),
            pltpu.SemaphoreType.DMA((DEPTH,)),
        ],
        compiler_params=pltpu.CompilerParams(
            needs_layout_passes=False, use_tc_tiling_on_sc=False),
    )(idx_t, tab_rows, bias16)


def kernel(indices, tables, bias):
    idx_t = indices.astype(jnp.int32).T
    tab_pad = jnp.pad(tables, ((0, FPAD - F), (0, VPAD - V)))
    tab_rows = (
        tab_pad.reshape(FPAD // 8, 8, CTILES, ROWLEN)
        .transpose(0, 2, 1, 3)
        .reshape(NROWS64, LANES)
    )
    bias16 = jnp.broadcast_to(bias.astype(jnp.float32), (LANES,))
    return _lookup_sum(idx_t, tab_rows, bias16)
